# baseline (speedup 1.0000x reference)
import functools
import math

import jax
import jax.numpy as jnp
from jax import lax
from jax.experimental import pallas as pl
from jax.experimental.pallas import tpu as pltpu
from jax.experimental.pallas import tpu_sc as plsc

EMB = 64
SCALE = math.sqrt(EMB)
GSZ = 128
PITCH = 129


def _make_transpose_kernel(V):
    info = plsc.get_sparse_core_info()
    NC, NS = info.num_cores, info.num_subcores
    NW = NC * NS
    NT = V // GSZ
    n_tail = V - NT * GSZ
    n_loop = NT // NW + 2
    n_kk = n_loop // 2

    mesh = plsc.VectorSubcoreMesh(core_axis_name="c", subcore_axis_name="s")

    @functools.partial(
        pl.kernel,
        mesh=mesh,
        compiler_params=pltpu.CompilerParams(
            use_tc_tiling_on_sc=True, needs_layout_passes=False
        ),
        out_type=jax.ShapeDtypeStruct((V // 2, GSZ), jnp.float32),
        scratch_types=[
            pltpu.VMEM((EMB, GSZ), jnp.float32),
            pltpu.VMEM((EMB, GSZ), jnp.float32),
            pltpu.VMEM((EMB, PITCH), jnp.float32),
            pltpu.VMEM((EMB, PITCH), jnp.float32),
            pltpu.VMEM((n_tail // 2, GSZ), jnp.float32),
            pltpu.SemaphoreType.DMA,
            pltpu.SemaphoreType.DMA,
            pltpu.SemaphoreType.DMA,
            pltpu.SemaphoreType.DMA,
        ],
    )
    def k(tt_hbm, tail_hbm, ts_hbm, ib0, ib1, ow0, ow1, tbuf,
          g0, g1, s0, s1):
        wid = lax.axis_index("s") * NC + lax.axis_index("c")
        ibuf = (ib0, ib1)
        outw = (ow0, ow1)
        gsem = (g0, g1)
        ssem = (s0, s1)
        gdummy = ts_hbm.at[pl.ds(0, EMB)]

        def vt(kk):
            return kk * NW + wid

        def fire(kk, b):
            @pl.when(vt(kk) < NT)
            def _():
                pltpu.async_copy(
                    tt_hbm.at[:, pl.ds(vt(kk) * GSZ, GSZ)], ibuf[b], gsem[b]
                )

        iota16 = lax.iota(jnp.int32, 16)
        rvecs = [(iota16 + 16 * m) // 2 for m in range(GSZ // 16)]
        pvecs = [EMB * lax.rem(iota16 + 16 * m, 2) for m in range(GSZ // 16)]

        def scatter_pass(b):
            @plsc.parallel_loop(0, EMB, 1, unroll=2)
            def _(e):
                for m in range(GSZ // 16):
                    v = ibuf[b][e, pl.ds(16 * m, 16)] * SCALE
                    plsc.store_scatter(outw[b], [rvecs[m], pvecs[m] + e], v)

        fire(0, 0)

        def body(kk2, _):
            for b in range(2):
                kk = 2 * kk2 + b
                o = 1 - b
                fire(kk + 1, o)
                if b == 0:

                    @pl.when(jnp.logical_and(kk2 >= 1, vt(kk - 2) < NT))
                    def _():
                        pltpu.make_async_copy(
                            outw[b].at[:, pl.ds(0, GSZ)], gdummy, ssem[b]
                        ).wait()

                else:

                    @pl.when(jnp.logical_and(kk2 >= 1, vt(kk - 2) < NT))
                    def _():
                        pltpu.make_async_copy(
                            outw[b].at[:, pl.ds(0, GSZ)], gdummy, ssem[b]
                        ).wait()

                @pl.when(vt(kk) < NT)
                def _():
                    pltpu.make_async_copy(
                        tt_hbm.at[:, pl.ds(0, GSZ)], ibuf[b], gsem[b]
                    ).wait()
                    scatter_pass(b)
                    pltpu.async_copy(
                        outw[b].at[:, pl.ds(0, GSZ)],
                        ts_hbm.at[pl.ds(vt(kk) * (GSZ // 2), GSZ // 2)],
                        ssem[b],
                    )

            return 0

        lax.fori_loop(0, n_kk, body, 0)
        for kk in (n_loop - 2, n_loop - 1):

            @pl.when(vt(kk) < NT)
            def _():
                pltpu.make_async_copy(
                    outw[kk % 2].at[:, pl.ds(0, GSZ)], gdummy, ssem[kk % 2]
                ).wait()

        @pl.when(wid == 0)
        def _():
            pltpu.sync_copy(tail_hbm, tbuf)
            pltpu.sync_copy(tbuf, ts_hbm.at[pl.ds(NT * (GSZ // 2),
                                                  n_tail // 2)])

    return k


def _make_gather_kernel(batch, hist, V):
    info = plsc.get_sparse_core_info()
    NC, NS = info.num_cores, info.num_subcores
    NW = NC * NS
    n_btile = batch // GSZ
    n_pairs = hist * n_btile
    per_w = n_pairs // NW
    assert n_pairs % (2 * NW) == 0 and batch % GSZ == 0 and EMB % 16 == 0
    n_kk = per_w // 2

    mesh = plsc.VectorSubcoreMesh(core_axis_name="c", subcore_axis_name="s")

    @functools.partial(
        pl.kernel,
        mesh=mesh,
        compiler_params=pltpu.CompilerParams(
            use_tc_tiling_on_sc=False, needs_layout_passes=False
        ),
        out_type=jax.ShapeDtypeStruct(
            (hist, EMB // 8, n_btile, 8, GSZ), jnp.float32
        ),
        scratch_types=[
            pltpu.VMEM((per_w, GSZ), jnp.int32),
            pltpu.VMEM((GSZ, EMB), jnp.float32),
            pltpu.VMEM((GSZ, EMB), jnp.float32),
            pltpu.VMEM((EMB // 8, 8, PITCH), jnp.float32),
            pltpu.VMEM((EMB // 8, 8, PITCH), jnp.float32),
            pltpu.SemaphoreType.DMA,
            pltpu.SemaphoreType.DMA,
            pltpu.SemaphoreType.DMA,
            pltpu.SemaphoreType.DMA,
        ],
    )
    def k(x_hbm, table_hbm, out_hbm, idx_v, rows0, rows1,
          ob0, ob1, g0, g1, s0, s1):
        wid = lax.axis_index("s") * NC + lax.axis_index("c")
        pltpu.sync_copy(x_hbm.at[wid], idx_v)

        rows = (rows0, rows1)
        obuf = (ob0, ob1)
        gsem = (g0, g1)
        ssem = (s0, s1)
        gdummy = out_hbm.at[0, :, 0]

        def fire(t, b):
            pltpu.async_copy(table_hbm.at[idx_v.at[t]], rows[b], gsem[b])

        iota16 = lax.iota(jnp.int32, 16)
        evecs = [(iota16 + 16 * j) // 8 for j in range(EMB // 16)]
        svecs = [lax.rem(iota16 + 16 * j, 8) for j in range(EMB // 16)]

        def scatter_pass(b):
            @plsc.parallel_loop(0, GSZ, 1, unroll=4)
            def _(l):
                colv = lax.broadcast(l, (16,))
                for j in range(EMB // 16):
                    v = rows[b][l, pl.ds(16 * j, 16)]
                    plsc.store_scatter(obuf[b], [evecs[j], svecs[j], colv], v)

        fire(0, 0)

        def body(kk, _):
            for b in range(2):
                cur = 2 * kk + b
                o = 1 - b
                if b == 0:
                    fire(cur + 1, o)
                else:

                    @pl.when(kk < n_kk - 1)
                    def _():
                        fire(cur + 1, o)

                pltpu.make_async_copy(
                    table_hbm.at[pl.ds(0, GSZ)], rows[b], gsem[b]
                ).wait()

                @pl.when(kk >= 1)
                def _():
                    pltpu.make_async_copy(
                        obuf[b].at[:, :, pl.ds(0, GSZ)], gdummy, ssem[b]
                    ).wait()

                scatter_pass(b)
                q = per_w * wid + cur
                h = 8 * (q // 256) + lax.rem(q, 8)
                bb = lax.rem(q, 256) // 8
                pltpu.async_copy(
                    obuf[b].at[:, :, pl.ds(0, GSZ)],
                    out_hbm.at[h, :, bb],
                    ssem[b],
                )
            return 0

        lax.fori_loop(0, n_kk, body, 0)
        pltpu.make_async_copy(
            ob0.at[:, :, pl.ds(0, GSZ)], gdummy, ssem[0]
        ).wait()
        pltpu.make_async_copy(
            ob1.at[:, :, pl.ds(0, GSZ)], gdummy, ssem[1]
        ).wait()

    def run(x, ts):
        x4 = x.reshape(batch // GSZ, GSZ, hist // 8, 8)
        x4 = jnp.transpose(x4, (2, 0, 3, 1))
        xt = x4.reshape(NW, per_w, GSZ)
        t64 = ts.reshape(V, EMB)
        return k(xt, t64)

    return run


def kernel(x, table):
    batch, hist = x.shape
    V = table.shape[0]
    NT = V // GSZ
    tail = (table[NT * GSZ:, :] * SCALE).reshape((V - NT * GSZ) // 2,
                                                 2 * EMB)
    ts = _make_transpose_kernel(V)(table.T, tail)
    o5 = _make_gather_kernel(batch, hist, V)(x.astype(jnp.int32), ts)
    return jnp.transpose(o5, (2, 4, 0, 1, 3)).reshape(batch, hist, EMB)


# speedup vs baseline: 1.0675x; 1.0675x over previous
import functools
import math

import jax
import jax.numpy as jnp
from jax import lax
from jax.experimental import pallas as pl
from jax.experimental.pallas import tpu as pltpu
from jax.experimental.pallas import tpu_sc as plsc

EMB = 64
SCALE = math.sqrt(EMB)
GSZ = 128
PITCH = 129


def _make_transpose_kernel(V):
    info = plsc.get_sparse_core_info()
    NC, NS = info.num_cores, info.num_subcores
    NW = NC * NS
    NT = V // GSZ
    n_tail = V - NT * GSZ
    n_loop = NT // NW + 2
    n_kk = n_loop // 2

    mesh = plsc.VectorSubcoreMesh(core_axis_name="c", subcore_axis_name="s")

    @functools.partial(
        pl.kernel,
        mesh=mesh,
        compiler_params=pltpu.CompilerParams(
            use_tc_tiling_on_sc=True, needs_layout_passes=False
        ),
        out_type=jax.ShapeDtypeStruct((V // 2, GSZ), jnp.float32),
        scratch_types=[
            pltpu.VMEM((EMB, PITCH), jnp.float32),
            pltpu.VMEM((EMB, PITCH), jnp.float32),
            pltpu.VMEM((GSZ // 2, PITCH), jnp.float32),
            pltpu.VMEM((GSZ // 2, PITCH), jnp.float32),
            pltpu.VMEM((n_tail // 2, GSZ), jnp.float32),
            pltpu.SemaphoreType.DMA,
            pltpu.SemaphoreType.DMA,
            pltpu.SemaphoreType.DMA,
            pltpu.SemaphoreType.DMA,
        ],
    )
    def k(tt_hbm, tail_hbm, ts_hbm, ib0, ib1, ow0, ow1, tbuf,
          g0, g1, s0, s1):
        wid = lax.axis_index("s") * NC + lax.axis_index("c")
        ibuf = (ib0, ib1)
        outw = (ow0, ow1)
        gsem = (g0, g1)
        ssem = (s0, s1)
        gdummy = ts_hbm.at[pl.ds(0, EMB)]

        def vt(kk):
            return kk * NW + wid

        def fire(kk, b):
            @pl.when(vt(kk) < NT)
            def _():
                pltpu.async_copy(
                    tt_hbm.at[:, pl.ds(vt(kk) * GSZ, GSZ)],
                    ibuf[b].at[:, pl.ds(0, GSZ)],
                    gsem[b],
                )

        iota16 = lax.iota(jnp.int32, 16)
        evecs4 = [iota16 + 16 * jj for jj in range(EMB // 16)]

        def trans_pass(b):
            @plsc.parallel_loop(0, GSZ // 2, 1, unroll=2)
            def _(kp):
                c0 = lax.broadcast(2 * kp, (16,))
                c1 = lax.broadcast(2 * kp + 1, (16,))
                for j in range(2 * EMB // 16):
                    colv = c0 if j < EMB // 16 else c1
                    v = plsc.load_gather(
                        ibuf[b], [evecs4[j % (EMB // 16)], colv]
                    )
                    outw[b][kp, pl.ds(16 * j, 16)] = v * SCALE

        fire(0, 0)

        def body(kk2, _):
            for b in range(2):
                kk = 2 * kk2 + b
                o = 1 - b
                fire(kk + 1, o)
                if b == 0:

                    @pl.when(jnp.logical_and(kk2 >= 1, vt(kk - 2) < NT))
                    def _():
                        pltpu.make_async_copy(
                            outw[b].at[:, pl.ds(0, GSZ)], gdummy, ssem[b]
                        ).wait()

                else:

                    @pl.when(jnp.logical_and(kk2 >= 1, vt(kk - 2) < NT))
                    def _():
                        pltpu.make_async_copy(
                            outw[b].at[:, pl.ds(0, GSZ)], gdummy, ssem[b]
                        ).wait()

                @pl.when(vt(kk) < NT)
                def _():
                    pltpu.make_async_copy(
                        tt_hbm.at[:, pl.ds(0, GSZ)],
                        ibuf[b].at[:, pl.ds(0, GSZ)],
                        gsem[b],
                    ).wait()
                    trans_pass(b)
                    pltpu.async_copy(
                        outw[b].at[:, pl.ds(0, GSZ)],
                        ts_hbm.at[pl.ds(vt(kk) * (GSZ // 2), GSZ // 2)],
                        ssem[b],
                    )

            return 0

        lax.fori_loop(0, n_kk, body, 0)
        for kk in (n_loop - 2, n_loop - 1):

            @pl.when(vt(kk) < NT)
            def _():
                pltpu.make_async_copy(
                    outw[kk % 2].at[:, pl.ds(0, GSZ)], gdummy, ssem[kk % 2]
                ).wait()

        @pl.when(wid == 0)
        def _():
            pltpu.sync_copy(tail_hbm, tbuf)
            pltpu.sync_copy(tbuf, ts_hbm.at[pl.ds(NT * (GSZ // 2),
                                                  n_tail // 2)])

    return k


def _make_gather_kernel(batch, hist, V):
    info = plsc.get_sparse_core_info()
    NC, NS = info.num_cores, info.num_subcores
    NW = NC * NS
    n_btile = batch // GSZ
    n_pairs = hist * n_btile
    per_w = n_pairs // NW
    assert n_pairs % (2 * NW) == 0 and batch % GSZ == 0 and EMB % 16 == 0
    n_kk = per_w // 2

    mesh = plsc.VectorSubcoreMesh(core_axis_name="c", subcore_axis_name="s")

    @functools.partial(
        pl.kernel,
        mesh=mesh,
        compiler_params=pltpu.CompilerParams(
            use_tc_tiling_on_sc=False, needs_layout_passes=False
        ),
        out_type=jax.ShapeDtypeStruct(
            (hist, EMB // 8, n_btile, 8, GSZ), jnp.float32
        ),
        scratch_types=[
            pltpu.VMEM((per_w, GSZ), jnp.int32),
            pltpu.VMEM((GSZ, EMB), jnp.float32),
            pltpu.VMEM((GSZ, EMB), jnp.float32),
            pltpu.VMEM((EMB // 8, 8, PITCH), jnp.float32),
            pltpu.VMEM((EMB // 8, 8, PITCH), jnp.float32),
            pltpu.SemaphoreType.DMA,
            pltpu.SemaphoreType.DMA,
            pltpu.SemaphoreType.DMA,
            pltpu.SemaphoreType.DMA,
        ],
    )
    def k(x_hbm, table_hbm, out_hbm, idx_v, rows0, rows1,
          ob0, ob1, g0, g1, s0, s1):
        wid = lax.axis_index("s") * NC + lax.axis_index("c")
        pltpu.sync_copy(x_hbm.at[wid], idx_v)

        rows = (rows0, rows1)
        obuf = (ob0, ob1)
        gsem = (g0, g1)
        ssem = (s0, s1)
        gdummy = out_hbm.at[0, :, 0]

        def fire(t, b):
            pltpu.async_copy(table_hbm.at[idx_v.at[t]], rows[b], gsem[b])

        iota16 = lax.iota(jnp.int32, 16)
        evecs = [(iota16 + 16 * j) // 8 for j in range(EMB // 16)]
        svecs = [lax.rem(iota16 + 16 * j, 8) for j in range(EMB // 16)]

        def scatter_pass(b):
            @plsc.parallel_loop(0, GSZ, 1, unroll=4)
            def _(l):
                colv = lax.broadcast(l, (16,))
                for j in range(EMB // 16):
                    v = rows[b][l, pl.ds(16 * j, 16)]
                    plsc.store_scatter(obuf[b], [evecs[j], svecs[j], colv], v)

        fire(0, 0)

        def body(kk, _):
            for b in range(2):
                cur = 2 * kk + b
                o = 1 - b
                if b == 0:
                    fire(cur + 1, o)
                else:

                    @pl.when(kk < n_kk - 1)
                    def _():
                        fire(cur + 1, o)

                pltpu.make_async_copy(
                    table_hbm.at[pl.ds(0, GSZ)], rows[b], gsem[b]
                ).wait()

                @pl.when(kk >= 1)
                def _():
                    pltpu.make_async_copy(
                        obuf[b].at[:, :, pl.ds(0, GSZ)], gdummy, ssem[b]
                    ).wait()

                scatter_pass(b)
                q = per_w * wid + cur
                h = 8 * (q // 256) + lax.rem(q, 8)
                bb = lax.rem(q, 256) // 8
                pltpu.async_copy(
                    obuf[b].at[:, :, pl.ds(0, GSZ)],
                    out_hbm.at[h, :, bb],
                    ssem[b],
                )
            return 0

        lax.fori_loop(0, n_kk, body, 0)
        pltpu.make_async_copy(
            ob0.at[:, :, pl.ds(0, GSZ)], gdummy, ssem[0]
        ).wait()
        pltpu.make_async_copy(
            ob1.at[:, :, pl.ds(0, GSZ)], gdummy, ssem[1]
        ).wait()

    def run(x, ts):
        x4 = x.reshape(batch // GSZ, GSZ, hist // 8, 8)
        x4 = jnp.transpose(x4, (2, 0, 3, 1))
        xt = x4.reshape(NW, per_w, GSZ)
        t64 = ts.reshape(V, EMB)
        return k(xt, t64)

    return run


def kernel(x, table):
    batch, hist = x.shape
    V = table.shape[0]
    NT = V // GSZ
    tail = (table[NT * GSZ:, :] * SCALE).reshape((V - NT * GSZ) // 2,
                                                 2 * EMB)
    ts = _make_transpose_kernel(V)(table.T, tail)
    o5 = _make_gather_kernel(batch, hist, V)(x.astype(jnp.int32), ts)
    return jnp.transpose(o5, (2, 4, 0, 1, 3)).reshape(batch, hist, EMB)


# speedup vs baseline: 1.0696x; 1.0019x over previous
import functools
import math

import jax
import jax.numpy as jnp
from jax import lax
from jax.experimental import pallas as pl
from jax.experimental.pallas import tpu as pltpu
from jax.experimental.pallas import tpu_sc as plsc

EMB = 64
SCALE = math.sqrt(EMB)
GSZ = 128
PITCH = 129


def _make_transpose_kernel(V):
    info = plsc.get_sparse_core_info()
    NC, NS = info.num_cores, info.num_subcores
    NW = NC * NS
    NT = V // GSZ
    n_tail = V - NT * GSZ
    n_loop = NT // NW + 2
    n_kk = n_loop // 2

    mesh = plsc.VectorSubcoreMesh(core_axis_name="c", subcore_axis_name="s")

    @functools.partial(
        pl.kernel,
        mesh=mesh,
        compiler_params=pltpu.CompilerParams(
            use_tc_tiling_on_sc=True, needs_layout_passes=False
        ),
        out_type=jax.ShapeDtypeStruct((V // 2, GSZ), jnp.float32),
        scratch_types=[
            pltpu.VMEM((EMB, PITCH), jnp.float32),
            pltpu.VMEM((EMB, PITCH), jnp.float32),
            pltpu.VMEM((GSZ // 2, PITCH), jnp.float32),
            pltpu.VMEM((GSZ // 2, PITCH), jnp.float32),
            pltpu.VMEM((n_tail // 2, GSZ), jnp.float32),
            pltpu.SemaphoreType.DMA,
            pltpu.SemaphoreType.DMA,
            pltpu.SemaphoreType.DMA,
            pltpu.SemaphoreType.DMA,
        ],
    )
    def k(tt_hbm, tail_hbm, ts_hbm, ib0, ib1, ow0, ow1, tbuf,
          g0, g1, s0, s1):
        wid = lax.axis_index("s") * NC + lax.axis_index("c")
        ibuf = (ib0, ib1)
        outw = (ow0, ow1)
        gsem = (g0, g1)
        ssem = (s0, s1)
        gdummy = ts_hbm.at[pl.ds(0, EMB)]

        def vt(kk):
            return kk * NW + wid

        def fire(kk, b):
            @pl.when(vt(kk) < NT)
            def _():
                pltpu.async_copy(
                    tt_hbm.at[:, pl.ds(vt(kk) * GSZ, GSZ)],
                    ibuf[b].at[:, pl.ds(0, GSZ)],
                    gsem[b],
                )

        iota16 = lax.iota(jnp.int32, 16)
        evecs4 = [iota16 + 16 * jj for jj in range(EMB // 16)]

        def trans_pass(b):
            @plsc.parallel_loop(0, GSZ // 2, 1, unroll=4)
            def _(kp):
                c0 = lax.broadcast(2 * kp, (16,))
                c1 = lax.broadcast(2 * kp + 1, (16,))
                for j in range(2 * EMB // 16):
                    colv = c0 if j < EMB // 16 else c1
                    v = plsc.load_gather(
                        ibuf[b], [evecs4[j % (EMB // 16)], colv]
                    )
                    outw[b][kp, pl.ds(16 * j, 16)] = v * SCALE

        fire(0, 0)

        def body(kk2, _):
            for b in range(2):
                kk = 2 * kk2 + b
                o = 1 - b
                fire(kk + 1, o)
                if b == 0:

                    @pl.when(jnp.logical_and(kk2 >= 1, vt(kk - 2) < NT))
                    def _():
                        pltpu.make_async_copy(
                            outw[b].at[:, pl.ds(0, GSZ)], gdummy, ssem[b]
                        ).wait()

                else:

                    @pl.when(jnp.logical_and(kk2 >= 1, vt(kk - 2) < NT))
                    def _():
                        pltpu.make_async_copy(
                            outw[b].at[:, pl.ds(0, GSZ)], gdummy, ssem[b]
                        ).wait()

                @pl.when(vt(kk) < NT)
                def _():
                    pltpu.make_async_copy(
                        tt_hbm.at[:, pl.ds(0, GSZ)],
                        ibuf[b].at[:, pl.ds(0, GSZ)],
                        gsem[b],
                    ).wait()
                    trans_pass(b)
                    pltpu.async_copy(
                        outw[b].at[:, pl.ds(0, GSZ)],
                        ts_hbm.at[pl.ds(vt(kk) * (GSZ // 2), GSZ // 2)],
                        ssem[b],
                    )

            return 0

        lax.fori_loop(0, n_kk, body, 0)
        for kk in (n_loop - 2, n_loop - 1):

            @pl.when(vt(kk) < NT)
            def _():
                pltpu.make_async_copy(
                    outw[kk % 2].at[:, pl.ds(0, GSZ)], gdummy, ssem[kk % 2]
                ).wait()

        @pl.when(wid == 0)
        def _():
            pltpu.sync_copy(tail_hbm, tbuf)
            pltpu.sync_copy(tbuf, ts_hbm.at[pl.ds(NT * (GSZ // 2),
                                                  n_tail // 2)])

    return k


def _make_gather_kernel(batch, hist, V):
    info = plsc.get_sparse_core_info()
    NC, NS = info.num_cores, info.num_subcores
    NW = NC * NS
    n_btile = batch // GSZ
    n_pairs = hist * n_btile
    per_w = n_pairs // NW
    assert n_pairs % (2 * NW) == 0 and batch % GSZ == 0 and EMB % 16 == 0
    n_kk = per_w // 2

    mesh = plsc.VectorSubcoreMesh(core_axis_name="c", subcore_axis_name="s")

    @functools.partial(
        pl.kernel,
        mesh=mesh,
        compiler_params=pltpu.CompilerParams(
            use_tc_tiling_on_sc=False, needs_layout_passes=False
        ),
        out_type=jax.ShapeDtypeStruct(
            (hist, EMB // 8, n_btile, 8, GSZ), jnp.float32
        ),
        scratch_types=[
            pltpu.VMEM((per_w, GSZ), jnp.int32),
            pltpu.VMEM((GSZ, EMB), jnp.float32),
            pltpu.VMEM((GSZ, EMB), jnp.float32),
            pltpu.VMEM((EMB // 8, 8, PITCH), jnp.float32),
            pltpu.VMEM((EMB // 8, 8, PITCH), jnp.float32),
            pltpu.SemaphoreType.DMA,
            pltpu.SemaphoreType.DMA,
            pltpu.SemaphoreType.DMA,
            pltpu.SemaphoreType.DMA,
        ],
    )
    def k(x_hbm, table_hbm, out_hbm, idx_v, rows0, rows1,
          ob0, ob1, g0, g1, s0, s1):
        wid = lax.axis_index("s") * NC + lax.axis_index("c")
        pltpu.sync_copy(x_hbm.at[wid], idx_v)

        rows = (rows0, rows1)
        obuf = (ob0, ob1)
        gsem = (g0, g1)
        ssem = (s0, s1)
        gdummy = out_hbm.at[0, :, 0]

        def fire(t, b):
            pltpu.async_copy(table_hbm.at[idx_v.at[t]], rows[b], gsem[b])

        iota16 = lax.iota(jnp.int32, 16)
        evecs = [(iota16 + 16 * j) // 8 for j in range(EMB // 16)]
        svecs = [lax.rem(iota16 + 16 * j, 8) for j in range(EMB // 16)]

        def scatter_pass(b):
            @plsc.parallel_loop(0, GSZ, 1, unroll=4)
            def _(l):
                colv = lax.broadcast(l, (16,))
                for j in range(EMB // 16):
                    v = rows[b][l, pl.ds(16 * j, 16)]
                    plsc.store_scatter(obuf[b], [evecs[j], svecs[j], colv], v)

        fire(0, 0)

        def body(kk, _):
            for b in range(2):
                cur = 2 * kk + b
                o = 1 - b
                if b == 0:
                    fire(cur + 1, o)
                else:

                    @pl.when(kk < n_kk - 1)
                    def _():
                        fire(cur + 1, o)

                pltpu.make_async_copy(
                    table_hbm.at[pl.ds(0, GSZ)], rows[b], gsem[b]
                ).wait()

                @pl.when(kk >= 1)
                def _():
                    pltpu.make_async_copy(
                        obuf[b].at[:, :, pl.ds(0, GSZ)], gdummy, ssem[b]
                    ).wait()

                scatter_pass(b)
                q = per_w * wid + cur
                h = 8 * (q // 256) + lax.rem(q, 8)
                bb = lax.rem(q, 256) // 8
                pltpu.async_copy(
                    obuf[b].at[:, :, pl.ds(0, GSZ)],
                    out_hbm.at[h, :, bb],
                    ssem[b],
                )
            return 0

        lax.fori_loop(0, n_kk, body, 0)
        pltpu.make_async_copy(
            ob0.at[:, :, pl.ds(0, GSZ)], gdummy, ssem[0]
        ).wait()
        pltpu.make_async_copy(
            ob1.at[:, :, pl.ds(0, GSZ)], gdummy, ssem[1]
        ).wait()

    def run(x, ts):
        x4 = x.reshape(batch // GSZ, GSZ, hist // 8, 8)
        x4 = jnp.transpose(x4, (2, 0, 3, 1))
        xt = x4.reshape(NW, per_w, GSZ)
        t64 = ts.reshape(V, EMB)
        return k(xt, t64)

    return run


def kernel(x, table):
    batch, hist = x.shape
    V = table.shape[0]
    NT = V // GSZ
    tail = (table[NT * GSZ:, :] * SCALE).reshape((V - NT * GSZ) // 2,
                                                 2 * EMB)
    ts = _make_transpose_kernel(V)(table.T, tail)
    o5 = _make_gather_kernel(batch, hist, V)(x.astype(jnp.int32), ts)
    return jnp.transpose(o5, (2, 4, 0, 1, 3)).reshape(batch, hist, EMB)


# speedup vs baseline: 1.3484x; 1.2606x over previous
import functools
import math

import jax
import jax.numpy as jnp
from jax import lax
from jax.experimental import pallas as pl
from jax.experimental.pallas import tpu as pltpu
from jax.experimental.pallas import tpu_sc as plsc

EMB = 64
SCALE = math.sqrt(EMB)
GSZ = 128
PITCH = 129


def _make_sc_kernel(batch, hist, V):
    info = plsc.get_sparse_core_info()
    NC, NS = info.num_cores, info.num_subcores
    NW = NC * NS
    n_btile = batch // GSZ
    n_pairs = hist * n_btile
    per_w = n_pairs // NW
    assert n_pairs % (2 * NW) == 0 and batch % GSZ == 0 and EMB % 16 == 0
    n_kk = per_w // 2

    mesh = plsc.VectorSubcoreMesh(core_axis_name="c", subcore_axis_name="s")

    @functools.partial(
        pl.kernel,
        mesh=mesh,
        compiler_params=pltpu.CompilerParams(
            use_tc_tiling_on_sc=False, needs_layout_passes=False
        ),
        out_type=jax.ShapeDtypeStruct(
            (hist, EMB // 8, n_btile, 8, GSZ), jnp.float32
        ),
        scratch_types=[
            pltpu.VMEM((per_w, GSZ), jnp.int32),
            pltpu.VMEM((GSZ, EMB), jnp.float32),
            pltpu.VMEM((GSZ, EMB), jnp.float32),
            pltpu.VMEM((EMB // 8, 8, PITCH), jnp.float32),
            pltpu.VMEM((EMB // 8, 8, PITCH), jnp.float32),
            pltpu.SemaphoreType.DMA,
            pltpu.SemaphoreType.DMA,
            pltpu.SemaphoreType.DMA,
            pltpu.SemaphoreType.DMA,
        ],
    )
    def k(x_hbm, table_hbm, out_hbm, idx_v, rows0, rows1,
          ob0, ob1, g0, g1, s0, s1):
        wid = lax.axis_index("s") * NC + lax.axis_index("c")
        pltpu.sync_copy(x_hbm.at[wid], idx_v)

        rows = (rows0, rows1)
        obuf = (ob0, ob1)
        gsem = (g0, g1)
        ssem = (s0, s1)
        gdummy = out_hbm.at[0, :, 0]

        def fire(t, b):
            pltpu.async_copy(table_hbm.at[idx_v.at[t]], rows[b], gsem[b])

        iota16 = lax.iota(jnp.int32, 16)
        evecs = [
            (iota16 + 16 * j) // 8 for j in range(EMB // 16)
        ]
        svecs = [
            lax.rem(iota16 + 16 * j, 8) for j in range(EMB // 16)
        ]

        def scatter_pass(b):
            @plsc.parallel_loop(0, GSZ, 1, unroll=4)
            def _(l):
                colv = lax.broadcast(l, (16,))
                for j in range(EMB // 16):
                    v = rows[b][l, pl.ds(16 * j, 16)] * SCALE
                    plsc.store_scatter(obuf[b], [evecs[j], svecs[j], colv], v)

        fire(0, 0)

        def body(kk, _):
            for b in range(2):
                cur = 2 * kk + b
                o = 1 - b
                if b == 0:
                    fire(cur + 1, o)
                else:

                    @pl.when(kk < n_kk - 1)
                    def _():
                        fire(cur + 1, o)

                pltpu.make_async_copy(
                    table_hbm.at[pl.ds(0, GSZ)], rows[b], gsem[b]
                ).wait()

                @pl.when(kk >= 1)
                def _():
                    pltpu.make_async_copy(
                        obuf[b].at[:, :, pl.ds(0, GSZ)], gdummy, ssem[b]
                    ).wait()

                scatter_pass(b)
                q = per_w * wid + cur
                h = 8 * (q // 256) + lax.rem(q, 8)
                bb = lax.rem(q, 256) // 8
                pltpu.async_copy(
                    obuf[b].at[:, :, pl.ds(0, GSZ)],
                    out_hbm.at[h, :, bb],
                    ssem[b],
                )
            return 0

        lax.fori_loop(0, n_kk, body, 0)
        pltpu.make_async_copy(
            ob0.at[:, :, pl.ds(0, GSZ)], gdummy, ssem[0]
        ).wait()
        pltpu.make_async_copy(
            ob1.at[:, :, pl.ds(0, GSZ)], gdummy, ssem[1]
        ).wait()

    def run(x, table):
        x4 = x.reshape(batch // GSZ, GSZ, hist // 8, 8)
        x4 = jnp.transpose(x4, (2, 0, 3, 1))
        xt = x4.reshape(NW, per_w, GSZ)
        return k(xt, table)

    return run


def kernel(x, table):
    batch, hist = x.shape
    o5 = _make_sc_kernel(batch, hist, table.shape[0])(
        x.astype(jnp.int32), table
    )
    return jnp.transpose(o5, (2, 4, 0, 1, 3)).reshape(batch, hist, EMB)
